# baseline (speedup 1.0000x reference)
import functools

import jax
import jax.numpy as jnp
from jax import lax
from jax.experimental import pallas as pl
from jax.experimental.pallas import tpu as pltpu
from jax.experimental.pallas import tpu_sc as plsc

S = 100000
DS = 32
N = 524288
DB = 26
DE = 26
H = 32
CP = 128
EOFF = 64


_S_BLK = 2048


def _mlp_body(sft, wb1, bb1, wb2, bb2, we1, be1, we2, be2, tbl_out):
    xt = sft[...]
    dims = (((0,), (0,)), ((), ()))
    hb = jax.nn.silu(
        lax.dot_general(xt, wb1[...], dims,
                        preferred_element_type=jnp.float32) + bb1[...])
    b = jnp.dot(hb, wb2[...], preferred_element_type=jnp.float32) + bb2[...]
    he = jax.nn.silu(
        lax.dot_general(xt, we1[...], dims,
                        preferred_element_type=jnp.float32) + be1[...])
    e = jnp.dot(he, we2[...], preferred_element_type=jnp.float32) + be2[...]
    tbl_out[...] = jnp.concatenate([b, e], axis=1)


def _mlp_table(sft, Wb1, bb1, Wb2, bb2, We1, be1, We2, be2):
    Wb2p = jnp.pad(Wb2, ((0, 0), (0, EOFF - DB)))
    bb2p = jnp.pad(bb2, (0, EOFF - DB)).reshape(1, EOFF)
    We2p = jnp.pad(We2, ((0, 0), (0, EOFF - DE)))
    be2p = jnp.pad(be2, (0, EOFF - DE)).reshape(1, EOFF)
    bb1r = bb1.reshape(1, H)
    be1r = be1.reshape(1, H)

    grid = (S + _S_BLK - 1) // _S_BLK
    full = lambda i: (0, 0)
    return pl.pallas_call(
        _mlp_body,
        grid=(grid,),
        in_specs=[
            pl.BlockSpec((DS, _S_BLK), lambda i: (0, i)),
            pl.BlockSpec((DS, H), full),
            pl.BlockSpec((1, H), full),
            pl.BlockSpec((H, EOFF), full),
            pl.BlockSpec((1, EOFF), full),
            pl.BlockSpec((DS, H), full),
            pl.BlockSpec((1, H), full),
            pl.BlockSpec((H, EOFF), full),
            pl.BlockSpec((1, EOFF), full),
        ],
        out_specs=pl.BlockSpec((_S_BLK, CP), lambda i: (i, 0)),
        out_shape=jax.ShapeDtypeStruct((S, CP), jnp.float32),
    )(sft, Wb1, bb1r, Wb2p, bb2p, We1, be1r, We2p, be2p)


_NC = 2
_NS = 16
_NW = _NC * _NS
_ROWS_W = N // _NW
_CHUNK = 128
_NCHUNK = _ROWS_W // _CHUNK
_NGRP = _CHUNK // 16
_SEG = 4
_SEG_ROWS = _ROWS_W // _SEG


def _sc_fused(loc_ind, tbl, bft, eft, csm, alert):
    mesh = plsc.VectorSubcoreMesh(core_axis_name="c", subcore_axis_name="s")

    @functools.partial(
        pl.kernel,
        mesh=mesh,
        out_type=(
            jax.ShapeDtypeStruct((N,), jnp.float32),
            jax.ShapeDtypeStruct((N,), jnp.float32),
            jax.ShapeDtypeStruct((N,), jnp.float32),
        ),
        scratch_types=[
            pltpu.VMEM((_CHUNK,), jnp.int32),
            pltpu.VMEM((_CHUNK,), jnp.int32),
            pltpu.VMEM((_CHUNK, CP), jnp.float32),
            pltpu.VMEM((_CHUNK, CP), jnp.float32),
            pltpu.VMEM((DB, _CHUNK), jnp.float32),
            pltpu.VMEM((DB, _CHUNK), jnp.float32),
            pltpu.VMEM((DE, _CHUNK), jnp.float32),
            pltpu.VMEM((DE, _CHUNK), jnp.float32),
            pltpu.VMEM((_CHUNK,), jnp.float32),
            pltpu.VMEM((_CHUNK,), jnp.float32),
            pltpu.VMEM((_CHUNK,), jnp.float32),
            pltpu.VMEM((_CHUNK,), jnp.float32),
            pltpu.VMEM((_SEG_ROWS,), jnp.float32),
            pltpu.VMEM((_SEG_ROWS,), jnp.float32),
            pltpu.VMEM((_SEG_ROWS,), jnp.float32),
            pltpu.VMEM((DB, 16), jnp.int32),
            pltpu.VMEM((DB, 16), jnp.int32),
            pltpu.SemaphoreType.DMA,
            pltpu.SemaphoreType.DMA,
            pltpu.SemaphoreType.DMA,
            pltpu.SemaphoreType.DMA,
            pltpu.SemaphoreType.DMA,
            pltpu.SemaphoreType.DMA,
        ],
        compiler_params=pltpu.CompilerParams(needs_layout_passes=False),
    )
    def k(idx_hbm, tbl_hbm, bft_hbm, eft_hbm, csm_hbm, al_hbm,
          eff_hbm, base_hbm, outc_hbm,
          idx_a, idx_b, rows_a, rows_b, bft_a, bft_b, eft_a, eft_b,
          csm_a, csm_b, al_a, al_b, eff_w, base_w, outc_w, jsv, jsev,
          sidx_a, sidx_b, sin_a, sin_b, sg_a, sg_b):
        wid = lax.axis_index("s") * _NC + lax.axis_index("c")
        wbase = wid * _ROWS_W
        lane = lax.iota(jnp.int32, 16)
        last = _NCHUNK - 1
        for d in range(DB):
            js = lax.rem(lane + d, jnp.full((16,), DB, jnp.int32))
            jsv[d, :] = js
            jsev[d, :] = js + EOFF

        def issue_inputs(c, idx_v, bft_v, eft_v, csm_v, al_v, sidx, sin):
            base = wbase + c * _CHUNK
            pltpu.async_copy(idx_hbm.at[pl.ds(base, _CHUNK)], idx_v, sidx)
            pltpu.async_copy(bft_hbm.at[:, pl.ds(base, _CHUNK)], bft_v, sin)
            pltpu.async_copy(eft_hbm.at[:, pl.ds(base, _CHUNK)], eft_v, sin)
            pltpu.async_copy(csm_hbm.at[pl.ds(base, _CHUNK)], csm_v, sin)
            pltpu.async_copy(al_hbm.at[pl.ds(base, _CHUNK)], al_v, sin)

        def wait_inputs(c, idx_v, bft_v, eft_v, csm_v, al_v, sidx, sin,
                        idx_only):
            base = wbase + c * _CHUNK
            if idx_only:
                pltpu.make_async_copy(
                    idx_hbm.at[pl.ds(base, _CHUNK)], idx_v, sidx).wait()
            else:
                pltpu.make_async_copy(
                    bft_hbm.at[:, pl.ds(base, _CHUNK)], bft_v, sin).wait()
                pltpu.make_async_copy(
                    eft_hbm.at[:, pl.ds(base, _CHUNK)], eft_v, sin).wait()
                pltpu.make_async_copy(
                    csm_hbm.at[pl.ds(base, _CHUNK)], csm_v, sin).wait()
                pltpu.make_async_copy(
                    al_hbm.at[pl.ds(base, _CHUNK)], al_v, sin).wait()

        def compute(c, half_off, rows_v, bft_v, eft_v, csm_v, al_v):
            off = c * _CHUNK - half_off

            @plsc.parallel_loop(0, _NGRP, 1)
            def group(g):
                r = g * 16 + lane
                acc_b0 = jnp.zeros((16,), jnp.float32)
                acc_b1 = jnp.zeros((16,), jnp.float32)
                acc_e0 = jnp.zeros((16,), jnp.float32)
                acc_e1 = jnp.zeros((16,), jnp.float32)
                for d in range(DB):
                    js = jsv[d, :]
                    jse = jsev[d, :]
                    pb = (plsc.load_gather(rows_v, [r, js])
                          * plsc.load_gather(bft_v, [js, r]))
                    pe = (plsc.load_gather(rows_v, [r, jse])
                          * plsc.load_gather(eft_v, [js, r]))
                    if d % 2 == 0:
                        acc_b0 += pb
                        acc_e0 += pe
                    else:
                        acc_b1 += pb
                        acc_e1 += pe
                acc_b = acc_b0 + acc_b1
                acc_e = acc_e0 + acc_e1
                baseline = jnp.minimum(jnp.exp(acc_b), 1e6)
                eff = 1.0 / (1.0 + jnp.exp(4.0 - acc_e))
                eff = jnp.clip(eff, 1e-6, 1.0 - 1e-6)
                sl = pl.ds(g * 16, 16)
                wsl = pl.ds(off + g * 16, 16)
                csm16 = csm_v[sl]
                al16 = al_v[sl]
                eff_w[wsl] = eff
                base_w[wsl] = baseline
                outc_w[wsl] = csm16 * baseline * (1.0 - al16 * eff)

        def issue_gather(rows_v, idx_v, sg):
            pltpu.async_copy(tbl_hbm.at[idx_v], rows_v, sg)

        def wait_gather(rows_v, idx_v, sg):
            pltpu.make_async_copy(tbl_hbm.at[idx_v], rows_v, sg).wait()

        issue_inputs(0, idx_a, bft_a, eft_a, csm_a, al_a, sidx_a, sin_a)
        issue_inputs(1, idx_b, bft_b, eft_b, csm_b, al_b, sidx_b, sin_b)
        wait_inputs(0, idx_a, bft_a, eft_a, csm_a, al_a, sidx_a, sin_a, True)
        issue_gather(rows_a, idx_a, sg_a)

        def pair(i, carry, half_off):
            c0 = 2 * i
            c1 = 2 * i + 1
            wait_inputs(c1, idx_b, bft_b, eft_b, csm_b, al_b,
                        sidx_b, sin_b, True)
            issue_gather(rows_b, idx_b, sg_b)
            wait_gather(rows_a, idx_a, sg_a)
            wait_inputs(c0, idx_a, bft_a, eft_a, csm_a, al_a,
                        sidx_a, sin_a, False)
            compute(c0, half_off, rows_a, bft_a, eft_a, csm_a, al_a)
            issue_inputs(jnp.minimum(c0 + 2, last), idx_a, bft_a, eft_a,
                         csm_a, al_a, sidx_a, sin_a)
            wait_gather(rows_b, idx_b, sg_b)
            wait_inputs(c1, idx_b, bft_b, eft_b, csm_b, al_b,
                        sidx_b, sin_b, False)
            compute(c1, half_off, rows_b, bft_b, eft_b, csm_b, al_b)
            wait_inputs(jnp.minimum(c0 + 2, last), idx_a, bft_a, eft_a,
                        csm_a, al_a, sidx_a, sin_a, True)
            issue_gather(rows_a, idx_a, sg_a)
            issue_inputs(jnp.minimum(c1 + 2, last), idx_b, bft_b, eft_b,
                         csm_b, al_b, sidx_b, sin_b)
            return carry

        pairs_per_seg = _NCHUNK // 2 // _SEG
        for seg in range(_SEG):
            lax.fori_loop(seg * pairs_per_seg, (seg + 1) * pairs_per_seg,
                          lambda i, c, o=seg * _SEG_ROWS: pair(i, c, o), 0)
            sb = wbase + seg * _SEG_ROWS
            pltpu.sync_copy(eff_w, eff_hbm.at[pl.ds(sb, _SEG_ROWS)])
            pltpu.sync_copy(base_w, base_hbm.at[pl.ds(sb, _SEG_ROWS)])
            pltpu.sync_copy(outc_w, outc_hbm.at[pl.ds(sb, _SEG_ROWS)])

        wait_gather(rows_a, idx_a, sg_a)
        wait_inputs(last, idx_a, bft_a, eft_a, csm_a, al_a,
                    sidx_a, sin_a, False)
        wait_inputs(last, idx_b, bft_b, eft_b, csm_b, al_b,
                    sidx_b, sin_b, True)
        wait_inputs(last, idx_b, bft_b, eft_b, csm_b, al_b,
                    sidx_b, sin_b, False)

    return k(loc_ind, tbl, bft, eft, csm, alert)


def kernel(hosps, loc_ind, county_summer_mean, alert, baseline_features,
           eff_features, index, spatial_features,
           Wb1, bb1, Wb2, bb2, We1, be1, We2, be2):
    tbl = _mlp_table(spatial_features.T, Wb1, bb1, Wb2, bb2,
                     We1, be1, We2, be2)
    eff, base, outc = _sc_fused(loc_ind, tbl, baseline_features.T,
                                eff_features.T, county_summer_mean, alert)
    return jnp.stack([eff, base, outc], axis=1)


# speedup vs baseline: 1.3913x; 1.3913x over previous
import functools

import jax
import jax.numpy as jnp
from jax import lax
from jax.experimental import pallas as pl
from jax.experimental.pallas import tpu as pltpu
from jax.experimental.pallas import tpu_sc as plsc

S = 100000
DS = 32
N = 524288
DB = 26
DE = 26
H = 32
CP = 128
EOFF = 64


_S_BLK = 2048


def _mlp_body(sft, wb1, bb1, wb2, bb2, we1, be1, we2, be2, tbl_out):
    xt = sft[...]
    dims = (((0,), (0,)), ((), ()))
    hb = jax.nn.silu(
        lax.dot_general(xt, wb1[...], dims,
                        preferred_element_type=jnp.float32) + bb1[...])
    b = jnp.dot(hb, wb2[...], preferred_element_type=jnp.float32) + bb2[...]
    he = jax.nn.silu(
        lax.dot_general(xt, we1[...], dims,
                        preferred_element_type=jnp.float32) + be1[...])
    e = jnp.dot(he, we2[...], preferred_element_type=jnp.float32) + be2[...]
    tbl_out[...] = jnp.concatenate([b, e], axis=1)


def _mlp_table(sft, Wb1, bb1, Wb2, bb2, We1, be1, We2, be2):
    Wb2p = jnp.pad(Wb2, ((0, 0), (0, EOFF - DB)))
    bb2p = jnp.pad(bb2, (0, EOFF - DB)).reshape(1, EOFF)
    We2p = jnp.pad(We2, ((0, 0), (0, EOFF - DE)))
    be2p = jnp.pad(be2, (0, EOFF - DE)).reshape(1, EOFF)
    bb1r = bb1.reshape(1, H)
    be1r = be1.reshape(1, H)

    grid = (S + _S_BLK - 1) // _S_BLK
    full = lambda i: (0, 0)
    return pl.pallas_call(
        _mlp_body,
        grid=(grid,),
        in_specs=[
            pl.BlockSpec((DS, _S_BLK), lambda i: (0, i)),
            pl.BlockSpec((DS, H), full),
            pl.BlockSpec((1, H), full),
            pl.BlockSpec((H, EOFF), full),
            pl.BlockSpec((1, EOFF), full),
            pl.BlockSpec((DS, H), full),
            pl.BlockSpec((1, H), full),
            pl.BlockSpec((H, EOFF), full),
            pl.BlockSpec((1, EOFF), full),
        ],
        out_specs=pl.BlockSpec((_S_BLK, CP), lambda i: (i, 0)),
        out_shape=jax.ShapeDtypeStruct((S, CP), jnp.float32),
    )(sft, Wb1, bb1r, Wb2p, bb2p, We1, be1r, We2p, be2p)


_NC = 2
_NS = 16
_NW = _NC * _NS
_ROWS_W = N // _NW
_CHUNK = 128
_NCHUNK = _ROWS_W // _CHUNK
_NGRP = _CHUNK // 16


def _sc_fused(loc_ind, tbl, bft, eft, csm, alert):
    mesh = plsc.VectorSubcoreMesh(core_axis_name="c", subcore_axis_name="s")

    @functools.partial(
        pl.kernel,
        mesh=mesh,
        out_type=(
            jax.ShapeDtypeStruct((N,), jnp.float32),
            jax.ShapeDtypeStruct((N,), jnp.float32),
            jax.ShapeDtypeStruct((N,), jnp.float32),
        ),
        scratch_types=[
            pltpu.VMEM((_CHUNK,), jnp.int32),
            pltpu.VMEM((_CHUNK,), jnp.int32),
            pltpu.VMEM((_CHUNK, CP), jnp.float32),
            pltpu.VMEM((_CHUNK, CP), jnp.float32),
            pltpu.VMEM((DB, _CHUNK), jnp.float32),
            pltpu.VMEM((DB, _CHUNK), jnp.float32),
            pltpu.VMEM((DE, _CHUNK), jnp.float32),
            pltpu.VMEM((DE, _CHUNK), jnp.float32),
            pltpu.VMEM((_CHUNK,), jnp.float32),
            pltpu.VMEM((_CHUNK,), jnp.float32),
            pltpu.VMEM((_CHUNK,), jnp.float32),
            pltpu.VMEM((_CHUNK,), jnp.float32),
            pltpu.VMEM((_ROWS_W,), jnp.float32),
            pltpu.VMEM((_ROWS_W,), jnp.float32),
            pltpu.VMEM((_ROWS_W,), jnp.float32),
            pltpu.SemaphoreType.DMA,
            pltpu.SemaphoreType.DMA,
            pltpu.SemaphoreType.DMA,
            pltpu.SemaphoreType.DMA,
            pltpu.SemaphoreType.DMA,
            pltpu.SemaphoreType.DMA,
        ],
        compiler_params=pltpu.CompilerParams(needs_layout_passes=False),
    )
    def k(idx_hbm, tbl_hbm, bft_hbm, eft_hbm, csm_hbm, al_hbm,
          eff_hbm, base_hbm, outc_hbm,
          idx_a, idx_b, rows_a, rows_b, bft_a, bft_b, eft_a, eft_b,
          csm_a, csm_b, al_a, al_b, eff_w, base_w, outc_w,
          sidx_a, sidx_b, sin_a, sin_b, sg_a, sg_b):
        wid = lax.axis_index("s") * _NC + lax.axis_index("c")
        wbase = wid * _ROWS_W
        lane = lax.iota(jnp.int32, 16)
        last = _NCHUNK - 1

        def issue_inputs(c, idx_v, bft_v, eft_v, csm_v, al_v, sidx, sin):
            base = wbase + c * _CHUNK
            pltpu.async_copy(idx_hbm.at[pl.ds(base, _CHUNK)], idx_v, sidx)
            pltpu.async_copy(bft_hbm.at[:, pl.ds(base, _CHUNK)], bft_v, sin)
            pltpu.async_copy(eft_hbm.at[:, pl.ds(base, _CHUNK)], eft_v, sin)
            pltpu.async_copy(csm_hbm.at[pl.ds(base, _CHUNK)], csm_v, sin)
            pltpu.async_copy(al_hbm.at[pl.ds(base, _CHUNK)], al_v, sin)

        def wait_inputs(c, idx_v, bft_v, eft_v, csm_v, al_v, sidx, sin,
                        idx_only):
            base = wbase + c * _CHUNK
            if idx_only:
                pltpu.make_async_copy(
                    idx_hbm.at[pl.ds(base, _CHUNK)], idx_v, sidx).wait()
            else:
                pltpu.make_async_copy(
                    bft_hbm.at[:, pl.ds(base, _CHUNK)], bft_v, sin).wait()
                pltpu.make_async_copy(
                    eft_hbm.at[:, pl.ds(base, _CHUNK)], eft_v, sin).wait()
                pltpu.make_async_copy(
                    csm_hbm.at[pl.ds(base, _CHUNK)], csm_v, sin).wait()
                pltpu.make_async_copy(
                    al_hbm.at[pl.ds(base, _CHUNK)], al_v, sin).wait()

        def compute(c, rows_v, bft_v, eft_v, csm_v, al_v):
            off = c * _CHUNK

            @plsc.parallel_loop(0, _NGRP, 1)
            def group(g):
                r = g * 16 + lane
                acc_b0 = jnp.zeros((16,), jnp.float32)
                acc_b1 = jnp.zeros((16,), jnp.float32)
                acc_e0 = jnp.zeros((16,), jnp.float32)
                acc_e1 = jnp.zeros((16,), jnp.float32)
                for d in range(DB):
                    js = lax.rem(lane + d, jnp.full((16,), DB, jnp.int32))
                    jse = js + EOFF
                    pb = (plsc.load_gather(rows_v, [r, js])
                          * plsc.load_gather(bft_v, [js, r]))
                    pe = (plsc.load_gather(rows_v, [r, jse])
                          * plsc.load_gather(eft_v, [js, r]))
                    if d % 2 == 0:
                        acc_b0 += pb
                        acc_e0 += pe
                    else:
                        acc_b1 += pb
                        acc_e1 += pe
                acc_b = acc_b0 + acc_b1
                acc_e = acc_e0 + acc_e1
                baseline = jnp.minimum(jnp.exp(acc_b), 1e6)
                eff = 1.0 / (1.0 + jnp.exp(4.0 - acc_e))
                eff = jnp.clip(eff, 1e-6, 1.0 - 1e-6)
                sl = pl.ds(g * 16, 16)
                wsl = pl.ds(off + g * 16, 16)
                csm16 = csm_v[sl]
                al16 = al_v[sl]
                eff_w[wsl] = eff
                base_w[wsl] = baseline
                outc_w[wsl] = csm16 * baseline * (1.0 - al16 * eff)

        def issue_gather(rows_v, idx_v, sg):
            pltpu.async_copy(tbl_hbm.at[idx_v], rows_v, sg)

        def wait_gather(rows_v, idx_v, sg):
            pltpu.make_async_copy(tbl_hbm.at[idx_v], rows_v, sg).wait()

        issue_inputs(0, idx_a, bft_a, eft_a, csm_a, al_a, sidx_a, sin_a)
        issue_inputs(1, idx_b, bft_b, eft_b, csm_b, al_b, sidx_b, sin_b)
        wait_inputs(0, idx_a, bft_a, eft_a, csm_a, al_a, sidx_a, sin_a, True)
        issue_gather(rows_a, idx_a, sg_a)

        def pair(i, carry):
            c0 = 2 * i
            c1 = 2 * i + 1
            wait_inputs(c1, idx_b, bft_b, eft_b, csm_b, al_b,
                        sidx_b, sin_b, True)
            issue_gather(rows_b, idx_b, sg_b)
            wait_gather(rows_a, idx_a, sg_a)
            wait_inputs(c0, idx_a, bft_a, eft_a, csm_a, al_a,
                        sidx_a, sin_a, False)
            compute(c0, rows_a, bft_a, eft_a, csm_a, al_a)
            issue_inputs(jnp.minimum(c0 + 2, last), idx_a, bft_a, eft_a,
                         csm_a, al_a, sidx_a, sin_a)
            wait_gather(rows_b, idx_b, sg_b)
            wait_inputs(c1, idx_b, bft_b, eft_b, csm_b, al_b,
                        sidx_b, sin_b, False)
            compute(c1, rows_b, bft_b, eft_b, csm_b, al_b)
            wait_inputs(jnp.minimum(c0 + 2, last), idx_a, bft_a, eft_a,
                        csm_a, al_a, sidx_a, sin_a, True)
            issue_gather(rows_a, idx_a, sg_a)
            issue_inputs(jnp.minimum(c1 + 2, last), idx_b, bft_b, eft_b,
                         csm_b, al_b, sidx_b, sin_b)
            return carry

        lax.fori_loop(0, _NCHUNK // 2, pair, 0)

        wait_gather(rows_a, idx_a, sg_a)
        wait_inputs(last, idx_a, bft_a, eft_a, csm_a, al_a,
                    sidx_a, sin_a, False)
        wait_inputs(last, idx_b, bft_b, eft_b, csm_b, al_b,
                    sidx_b, sin_b, True)
        wait_inputs(last, idx_b, bft_b, eft_b, csm_b, al_b,
                    sidx_b, sin_b, False)

        pltpu.sync_copy(eff_w, eff_hbm.at[pl.ds(wbase, _ROWS_W)])
        pltpu.sync_copy(base_w, base_hbm.at[pl.ds(wbase, _ROWS_W)])
        pltpu.sync_copy(outc_w, outc_hbm.at[pl.ds(wbase, _ROWS_W)])

    return k(loc_ind, tbl, bft, eft, csm, alert)


def kernel(hosps, loc_ind, county_summer_mean, alert, baseline_features,
           eff_features, index, spatial_features,
           Wb1, bb1, Wb2, bb2, We1, be1, We2, be2):
    tbl = _mlp_table(spatial_features.T, Wb1, bb1, Wb2, bb2,
                     We1, be1, We2, be2)
    eff, base, outc = _sc_fused(loc_ind, tbl, baseline_features.T,
                                eff_features.T, county_summer_mean, alert)
    return jnp.stack([eff, base, outc], axis=1)


# speedup vs baseline: 1.6253x; 1.1683x over previous
import functools

import jax
import jax.numpy as jnp
from jax import lax
from jax.experimental import pallas as pl
from jax.experimental.pallas import tpu as pltpu
from jax.experimental.pallas import tpu_sc as plsc

S = 100000
DS = 32
N = 524288
DB = 26
DE = 26
H = 32
CP = 128
EOFF = 64


_S_BLK = 2048


def _mlp_body(sft, wb1, bb1, wb2, bb2, we1, be1, we2, be2, tbl_out):
    xt = sft[...]
    dims = (((0,), (0,)), ((), ()))
    hb = jax.nn.silu(
        lax.dot_general(xt, wb1[...], dims,
                        preferred_element_type=jnp.float32) + bb1[...])
    b = jnp.dot(hb, wb2[...], preferred_element_type=jnp.float32) + bb2[...]
    he = jax.nn.silu(
        lax.dot_general(xt, we1[...], dims,
                        preferred_element_type=jnp.float32) + be1[...])
    e = jnp.dot(he, we2[...], preferred_element_type=jnp.float32) + be2[...]
    tbl_out[...] = jnp.concatenate([b, e], axis=1)


def _mlp_table(sft, Wb1, bb1, Wb2, bb2, We1, be1, We2, be2):
    Wb2p = jnp.pad(Wb2, ((0, 0), (0, EOFF - DB)))
    bb2p = jnp.pad(bb2, (0, EOFF - DB)).reshape(1, EOFF)
    We2p = jnp.pad(We2, ((0, 0), (0, EOFF - DE)))
    be2p = jnp.pad(be2, (0, EOFF - DE)).reshape(1, EOFF)
    bb1r = bb1.reshape(1, H)
    be1r = be1.reshape(1, H)

    grid = (S + _S_BLK - 1) // _S_BLK
    full = lambda i: (0, 0)
    return pl.pallas_call(
        _mlp_body,
        grid=(grid,),
        in_specs=[
            pl.BlockSpec((DS, _S_BLK), lambda i: (0, i)),
            pl.BlockSpec((DS, H), full),
            pl.BlockSpec((1, H), full),
            pl.BlockSpec((H, EOFF), full),
            pl.BlockSpec((1, EOFF), full),
            pl.BlockSpec((DS, H), full),
            pl.BlockSpec((1, H), full),
            pl.BlockSpec((H, EOFF), full),
            pl.BlockSpec((1, EOFF), full),
        ],
        out_specs=pl.BlockSpec((_S_BLK, CP), lambda i: (i, 0)),
        out_shape=jax.ShapeDtypeStruct((S, CP), jnp.float32),
    )(sft, Wb1, bb1r, Wb2p, bb2p, We1, be1r, We2p, be2p)


_NC = 2
_NS = 16
_NW = _NC * _NS
_ROWS_W = N // _NW
_CHUNK = 256
_NCHUNK = _ROWS_W // _CHUNK
_NGRP = _CHUNK // 16
_SEG = 8
_SEG_ROWS = _ROWS_W // _SEG


def _sc_fused(loc_ind, tbl, bft, eft, csm, alert):
    mesh = plsc.VectorSubcoreMesh(core_axis_name="c", subcore_axis_name="s")

    @functools.partial(
        pl.kernel,
        mesh=mesh,
        out_type=(
            jax.ShapeDtypeStruct((N,), jnp.float32),
            jax.ShapeDtypeStruct((N,), jnp.float32),
            jax.ShapeDtypeStruct((N,), jnp.float32),
        ),
        scratch_types=[
            pltpu.VMEM((_CHUNK,), jnp.int32),
            pltpu.VMEM((_CHUNK,), jnp.int32),
            pltpu.VMEM((_CHUNK, CP), jnp.float32),
            pltpu.VMEM((_CHUNK, CP), jnp.float32),
            pltpu.VMEM((DB, _CHUNK), jnp.float32),
            pltpu.VMEM((DB, _CHUNK), jnp.float32),
            pltpu.VMEM((DE, _CHUNK), jnp.float32),
            pltpu.VMEM((DE, _CHUNK), jnp.float32),
            pltpu.VMEM((_CHUNK,), jnp.float32),
            pltpu.VMEM((_CHUNK,), jnp.float32),
            pltpu.VMEM((_CHUNK,), jnp.float32),
            pltpu.VMEM((_CHUNK,), jnp.float32),
            pltpu.VMEM((_SEG_ROWS,), jnp.float32),
            pltpu.VMEM((_SEG_ROWS,), jnp.float32),
            pltpu.VMEM((_SEG_ROWS,), jnp.float32),
            pltpu.SemaphoreType.DMA,
            pltpu.SemaphoreType.DMA,
            pltpu.SemaphoreType.DMA,
            pltpu.SemaphoreType.DMA,
            pltpu.SemaphoreType.DMA,
            pltpu.SemaphoreType.DMA,
        ],
        compiler_params=pltpu.CompilerParams(needs_layout_passes=False),
    )
    def k(idx_hbm, tbl_hbm, bft_hbm, eft_hbm, csm_hbm, al_hbm,
          eff_hbm, base_hbm, outc_hbm,
          idx_a, idx_b, rows_a, rows_b, bft_a, bft_b, eft_a, eft_b,
          csm_a, csm_b, al_a, al_b, eff_w, base_w, outc_w,
          sidx_a, sidx_b, sin_a, sin_b, sg_a, sg_b):
        wid = lax.axis_index("s") * _NC + lax.axis_index("c")
        wbase = wid * _ROWS_W
        lane = lax.iota(jnp.int32, 16)
        last = _NCHUNK - 1

        def issue_inputs(c, idx_v, bft_v, eft_v, csm_v, al_v, sidx, sin):
            base = wbase + c * _CHUNK
            pltpu.async_copy(idx_hbm.at[pl.ds(base, _CHUNK)], idx_v, sidx)
            pltpu.async_copy(bft_hbm.at[:, pl.ds(base, _CHUNK)], bft_v, sin)
            pltpu.async_copy(eft_hbm.at[:, pl.ds(base, _CHUNK)], eft_v, sin)
            pltpu.async_copy(csm_hbm.at[pl.ds(base, _CHUNK)], csm_v, sin)
            pltpu.async_copy(al_hbm.at[pl.ds(base, _CHUNK)], al_v, sin)

        def wait_inputs(c, idx_v, bft_v, eft_v, csm_v, al_v, sidx, sin,
                        idx_only):
            base = wbase + c * _CHUNK
            if idx_only:
                pltpu.make_async_copy(
                    idx_hbm.at[pl.ds(base, _CHUNK)], idx_v, sidx).wait()
            else:
                pltpu.make_async_copy(
                    bft_hbm.at[:, pl.ds(base, _CHUNK)], bft_v, sin).wait()
                pltpu.make_async_copy(
                    eft_hbm.at[:, pl.ds(base, _CHUNK)], eft_v, sin).wait()
                pltpu.make_async_copy(
                    csm_hbm.at[pl.ds(base, _CHUNK)], csm_v, sin).wait()
                pltpu.make_async_copy(
                    al_hbm.at[pl.ds(base, _CHUNK)], al_v, sin).wait()

        def compute(c, seg_off, rows_v, bft_v, eft_v, csm_v, al_v):
            off = c * _CHUNK - seg_off

            @plsc.parallel_loop(0, _NGRP, 1)
            def group(g):
                r = g * 16 + lane
                acc_b0 = jnp.zeros((16,), jnp.float32)
                acc_b1 = jnp.zeros((16,), jnp.float32)
                acc_e0 = jnp.zeros((16,), jnp.float32)
                acc_e1 = jnp.zeros((16,), jnp.float32)
                for d in range(DB):
                    js = lax.rem(lane + d, jnp.full((16,), DB, jnp.int32))
                    jse = js + EOFF
                    pb = (plsc.load_gather(rows_v, [r, js])
                          * plsc.load_gather(bft_v, [js, r]))
                    pe = (plsc.load_gather(rows_v, [r, jse])
                          * plsc.load_gather(eft_v, [js, r]))
                    if d % 2 == 0:
                        acc_b0 += pb
                        acc_e0 += pe
                    else:
                        acc_b1 += pb
                        acc_e1 += pe
                acc_b = acc_b0 + acc_b1
                acc_e = acc_e0 + acc_e1
                baseline = jnp.minimum(jnp.exp(acc_b), 1e6)
                eff = 1.0 / (1.0 + jnp.exp(4.0 - acc_e))
                eff = jnp.clip(eff, 1e-6, 1.0 - 1e-6)
                sl = pl.ds(g * 16, 16)
                wsl = pl.ds(off + g * 16, 16)
                csm16 = csm_v[sl]
                al16 = al_v[sl]
                eff_w[wsl] = eff
                base_w[wsl] = baseline
                outc_w[wsl] = csm16 * baseline * (1.0 - al16 * eff)

        def issue_gather(rows_v, idx_v, sg):
            pltpu.async_copy(tbl_hbm.at[idx_v], rows_v, sg)

        def wait_gather(rows_v, idx_v, sg):
            pltpu.make_async_copy(tbl_hbm.at[idx_v], rows_v, sg).wait()

        issue_inputs(0, idx_a, bft_a, eft_a, csm_a, al_a, sidx_a, sin_a)
        issue_inputs(1, idx_b, bft_b, eft_b, csm_b, al_b, sidx_b, sin_b)
        wait_inputs(0, idx_a, bft_a, eft_a, csm_a, al_a, sidx_a, sin_a, True)
        issue_gather(rows_a, idx_a, sg_a)

        def pair(i, carry):
            seg_off = carry
            c0 = 2 * i
            c1 = 2 * i + 1
            wait_inputs(c1, idx_b, bft_b, eft_b, csm_b, al_b,
                        sidx_b, sin_b, True)
            issue_gather(rows_b, idx_b, sg_b)
            wait_gather(rows_a, idx_a, sg_a)
            wait_inputs(c0, idx_a, bft_a, eft_a, csm_a, al_a,
                        sidx_a, sin_a, False)
            compute(c0, seg_off, rows_a, bft_a, eft_a, csm_a, al_a)
            issue_inputs(jnp.minimum(c0 + 2, last), idx_a, bft_a, eft_a,
                         csm_a, al_a, sidx_a, sin_a)
            wait_gather(rows_b, idx_b, sg_b)
            wait_inputs(c1, idx_b, bft_b, eft_b, csm_b, al_b,
                        sidx_b, sin_b, False)
            compute(c1, seg_off, rows_b, bft_b, eft_b, csm_b, al_b)
            wait_inputs(jnp.minimum(c0 + 2, last), idx_a, bft_a, eft_a,
                        csm_a, al_a, sidx_a, sin_a, True)
            issue_gather(rows_a, idx_a, sg_a)
            issue_inputs(jnp.minimum(c1 + 2, last), idx_b, bft_b, eft_b,
                         csm_b, al_b, sidx_b, sin_b)
            return carry

        pairs_per_seg = _NCHUNK // 2 // _SEG

        def seg_body(s, carry):
            seg_off = s * _SEG_ROWS
            lax.fori_loop(s * pairs_per_seg, (s + 1) * pairs_per_seg,
                          pair, seg_off)
            sb = wbase + seg_off
            pltpu.sync_copy(eff_w, eff_hbm.at[pl.ds(sb, _SEG_ROWS)])
            pltpu.sync_copy(base_w, base_hbm.at[pl.ds(sb, _SEG_ROWS)])
            pltpu.sync_copy(outc_w, outc_hbm.at[pl.ds(sb, _SEG_ROWS)])
            return carry

        lax.fori_loop(0, _SEG, seg_body, 0)

        wait_gather(rows_a, idx_a, sg_a)
        wait_inputs(last, idx_a, bft_a, eft_a, csm_a, al_a,
                    sidx_a, sin_a, False)
        wait_inputs(last, idx_b, bft_b, eft_b, csm_b, al_b,
                    sidx_b, sin_b, True)
        wait_inputs(last, idx_b, bft_b, eft_b, csm_b, al_b,
                    sidx_b, sin_b, False)


    return k(loc_ind, tbl, bft, eft, csm, alert)


def kernel(hosps, loc_ind, county_summer_mean, alert, baseline_features,
           eff_features, index, spatial_features,
           Wb1, bb1, Wb2, bb2, We1, be1, We2, be2):
    tbl = _mlp_table(spatial_features.T, Wb1, bb1, Wb2, bb2,
                     We1, be1, We2, be2)
    eff, base, outc = _sc_fused(loc_ind, tbl, baseline_features.T,
                                eff_features.T, county_summer_mean, alert)
    return jnp.stack([eff, base, outc], axis=1)
